# baseline (speedup 1.0000x reference)
import functools

import jax
import jax.numpy as jnp
from jax import lax
from jax.experimental import pallas as pl
from jax.experimental.pallas import tpu as pltpu
from jax.experimental.pallas import tpu_sc as plsc

NUM_ENTITIES = 1000000
DIM = 64
PROP_DIM = 32
NUM_TYPES = 8
BATCH = 16384

_NC = 2
_NS = 16
_NW = _NC * _NS
_BPW = BATCH // _NW
_CHUNK = 128
_NCHUNK = _BPW // _CHUNK


def _sc_gather_body(idx_hbm, et_hbm, edi_hbm, pd_hbm, tbl_hbm,
                    types_out, data_out, unspec_out,
                    idx_v, didx_v, types_v, rows_v, urows_v,
                    sem_t, sem_d, sem_r, sem_u):
    wid = lax.axis_index("s") * _NC + lax.axis_index("c")
    base = wid * _BPW
    for j in range(_NCHUNK):
        pltpu.sync_copy(idx_hbm.at[pl.ds(base + j * _CHUNK, _CHUNK)],
                        idx_v.at[j])
    ct = [pltpu.async_copy(et_hbm.at[idx_v.at[j]], types_v.at[j], sem_t)
          for j in range(_NCHUNK)]
    cd = [pltpu.async_copy(edi_hbm.at[idx_v.at[j]], didx_v.at[j], sem_d)
          for j in range(_NCHUNK)]
    cu = [pltpu.async_copy(tbl_hbm.at[idx_v.at[j]], urows_v.at[j], sem_u)
          for j in range(_NCHUNK)]
    for c in cd:
        c.wait()
    cr = [pltpu.async_copy(pd_hbm.at[didx_v.at[j]], rows_v.at[j], sem_r)
          for j in range(_NCHUNK)]
    for c in ct:
        c.wait()
    for j in range(_NCHUNK):
        pltpu.sync_copy(types_v.at[j],
                        types_out.at[pl.ds(base + j * _CHUNK, _CHUNK)])
    for c in cu:
        c.wait()
    for j in range(_NCHUNK):
        pltpu.sync_copy(urows_v.at[j],
                        unspec_out.at[pl.ds(base + j * _CHUNK, _CHUNK)])
    for c in cr:
        c.wait()
    for j in range(_NCHUNK):
        pltpu.sync_copy(rows_v.at[j],
                        data_out.at[pl.ds(base + j * _CHUNK, _CHUNK)])


@functools.cache
def _sc_gather_call():
    return pl.kernel(
        _sc_gather_body,
        out_type=[
            jax.ShapeDtypeStruct((BATCH,), jnp.int32),
            jax.ShapeDtypeStruct((BATCH, PROP_DIM), jnp.float32),
            jax.ShapeDtypeStruct((BATCH, DIM), jnp.float32),
        ],
        mesh=plsc.VectorSubcoreMesh(core_axis_name="c", subcore_axis_name="s"),
        compiler_params=pltpu.CompilerParams(use_tc_tiling_on_sc=False),
        scratch_types=[
            pltpu.VMEM((_NCHUNK, _CHUNK), jnp.int32),
            pltpu.VMEM((_NCHUNK, _CHUNK), jnp.int32),
            pltpu.VMEM((_NCHUNK, _CHUNK), jnp.int32),
            pltpu.VMEM((_NCHUNK, _CHUNK, PROP_DIM), jnp.float32),
            pltpu.VMEM((_NCHUNK, _CHUNK, DIM), jnp.float32),
            pltpu.SemaphoreType.DMA,
            pltpu.SemaphoreType.DMA,
            pltpu.SemaphoreType.DMA,
            pltpu.SemaphoreType.DMA,
        ],
    )

_TBLK = 2048


def _tc_encode_body(types_ref, data_ref, unspec_ref, W_ref, b_ref, out_ref):
    t = types_ref[...]
    data = data_ref[...]
    tclip = jnp.minimum(t, NUM_TYPES - 1)
    acc = jnp.zeros((_TBLK, DIM), jnp.float32)
    for i in range(NUM_TYPES):
        enc = jnp.dot(data, W_ref[i], preferred_element_type=jnp.float32)
        enc = enc + b_ref[i][None, :]
        acc = jnp.where(tclip == i, enc, acc)
    out_ref[...] = jnp.where(t == NUM_TYPES, unspec_ref[...], jnp.tanh(acc))


def _tc_encode(types2d, data, unspec, W, b):
    nblk = BATCH // _TBLK
    return pl.pallas_call(
        _tc_encode_body,
        grid=(nblk,),
        in_specs=[
            pl.BlockSpec((_TBLK, 1), lambda i: (i, 0)),
            pl.BlockSpec((_TBLK, PROP_DIM), lambda i: (i, 0)),
            pl.BlockSpec((_TBLK, DIM), lambda i: (i, 0)),
            pl.BlockSpec((NUM_TYPES, PROP_DIM, DIM), lambda i: (0, 0, 0)),
            pl.BlockSpec((NUM_TYPES, DIM), lambda i: (0, 0)),
        ],
        out_specs=pl.BlockSpec((_TBLK, DIM), lambda i: (i, 0)),
        out_shape=jax.ShapeDtypeStruct((BATCH, DIM), jnp.float32),
    )(types2d, data, unspec, W, b)


def kernel(indices, entity_types, entity_data_idx, prop_data, W, b, table):
    idx = indices.astype(jnp.int32)
    et = entity_types.astype(jnp.int32)
    edi = entity_data_idx.astype(jnp.int32)
    types_b, data_b, unspec_b = _sc_gather_call()(idx, et, edi, prop_data, table)
    return _tc_encode(types_b.reshape(BATCH, 1), data_b, unspec_b, W, b)
